# baseline (speedup 1.0000x reference)
import functools

import jax
import jax.numpy as jnp
from jax import lax
from jax.experimental import pallas as pl
from jax.experimental.pallas import tpu as pltpu
from jax.experimental.pallas import tpu_sc as plsc

_NC = 2
_NS = 16
_L = 16
_NW = _NC * _NS


@functools.lru_cache(maxsize=None)
def _build(B, F, V):
    rows_w = B // _NW
    bf_w = rows_w * F
    n_chunks = rows_w // _L

    mesh = plsc.VectorSubcoreMesh(
        core_axis_name="c", subcore_axis_name="s",
        num_cores=_NC, num_subcores=_NS,
    )

    @functools.partial(
        pl.kernel,
        out_type=jax.ShapeDtypeStruct((B,), jnp.float32),
        mesh=mesh,
        scratch_types=[
            pltpu.VMEM((bf_w,), jnp.int32),
            pltpu.VMEM((bf_w,), jnp.float32),
            pltpu.VMEM((rows_w,), jnp.float32),
            pltpu.VMEM((_L,), jnp.float32),
            pltpu.SemaphoreType.DMA,
            pltpu.SemaphoreType.DMA,
        ],
    )
    def launch(idx_hbm, table_hbm, bias_hbm, out_hbm,
               idx_v, vals_v, out_v, bias_v, sem_i, sem_g):
        wid = lax.axis_index("s") * _NC + lax.axis_index("c")
        idx_copies = [
            pltpu.async_copy(
                idx_hbm.at[pl.ds(f * B + wid * rows_w, rows_w)],
                idx_v.at[pl.ds(f * rows_w, rows_w)],
                sem_i,
            )
            for f in range(F)
        ]
        pltpu.sync_copy(bias_hbm, bias_v)
        for c in idx_copies:
            c.wait()

        vregs_f = rows_w // _L

        def add_offsets(j, _):
            sl = pl.ds(j * _L, _L)
            idx_v[sl] = idx_v[sl] + (j // vregs_f) * V
            return _

        lax.fori_loop(0, bf_w // _L, add_offsets, None)

        gathers = [
            pltpu.async_copy(
                table_hbm.at[idx_v.at[pl.ds(f * rows_w, rows_w)]],
                vals_v.at[pl.ds(f * rows_w, rows_w)],
                sem_g,
            )
            for f in range(F)
        ]
        for g in gathers:
            g.wait()

        bias_vec = bias_v[...]
        for c in range(n_chunks):
            acc = bias_vec
            for f in range(F):
                acc = acc + vals_v[pl.ds(f * rows_w + c * _L, _L)]
            out_v[pl.ds(c * _L, _L)] = acc

        pltpu.sync_copy(out_v, out_hbm.at[pl.ds(wid * rows_w, rows_w)])

    return launch


def kernel(indices, weights, bias):
    B, F = indices.shape
    _, V, U = weights.shape
    table = jnp.concatenate([weights[f, :, 0] for f in range(F)])
    idx_t = indices.T.reshape(B * F)
    bias_vec = jnp.broadcast_to(bias.astype(jnp.float32), (_L,))
    out = _build(B, F, V)(idx_t, table, bias_vec)
    return out.reshape(B, U)


# speedup vs baseline: 3.0347x; 3.0347x over previous
import functools

import jax
import jax.numpy as jnp
from jax import lax
from jax.experimental import pallas as pl
from jax.experimental.pallas import tpu as pltpu
from jax.experimental.pallas import tpu_sc as plsc

_NC = 2
_NS = 16
_L = 16
_NW = _NC * _NS


@functools.lru_cache(maxsize=None)
def _build(B, F, V):
    rows_w = B // _NW
    bf_w = rows_w * F
    n_chunks = rows_w // _L

    mesh = plsc.VectorSubcoreMesh(
        core_axis_name="c", subcore_axis_name="s",
        num_cores=_NC, num_subcores=_NS,
    )

    @functools.partial(
        pl.kernel,
        out_type=jax.ShapeDtypeStruct((B,), jnp.float32),
        mesh=mesh,
        scratch_types=[
            pltpu.VMEM((bf_w,), jnp.int32),
            pltpu.VMEM((bf_w,), jnp.float32),
            pltpu.VMEM((rows_w,), jnp.float32),
            pltpu.VMEM((_L,), jnp.float32),
            pltpu.SemaphoreType.DMA,
            pltpu.SemaphoreType.DMA,
        ],
    )
    def launch(idx_hbm, table_hbm, bias_hbm, out_hbm,
               idx_v, vals_v, out_v, bias_v, sem_i, sem_g):
        wid = lax.axis_index("s") * _NC + lax.axis_index("c")
        idx_copies = [
            pltpu.async_copy(
                idx_hbm.at[pl.ds(f * B + wid * rows_w, rows_w)],
                idx_v.at[pl.ds(f * rows_w, rows_w)],
                sem_i,
            )
            for f in range(F)
        ]
        pltpu.sync_copy(bias_hbm, bias_v)
        for c in idx_copies:
            c.wait()

        vregs_f = rows_w // _L

        def add_offsets(j, _):
            sl = pl.ds(j * _L, _L)
            idx_v[sl] = idx_v[sl] + (j // vregs_f) * V
            return _

        lax.fori_loop(0, bf_w // _L, add_offsets, None)

        gathers = [
            pltpu.async_copy(
                table_hbm.at[idx_v.at[pl.ds(f * rows_w, rows_w)]],
                vals_v.at[pl.ds(f * rows_w, rows_w)],
                sem_g,
            )
            for f in range(F)
        ]
        for g in gathers:
            g.wait()

        bias_vec = bias_v[...]
        for c in range(n_chunks):
            acc = bias_vec
            for f in range(F):
                acc = acc + vals_v[pl.ds(f * rows_w + c * _L, _L)]
            out_v[pl.ds(c * _L, _L)] = acc

        pltpu.sync_copy(out_v, out_hbm.at[pl.ds(wid * rows_w, rows_w)])

    return launch


def kernel(indices, weights, bias):
    B, F = indices.shape
    _, V, U = weights.shape
    table = lax.optimization_barrier(weights.squeeze(-1)).reshape(F * V)
    idx_t = indices.T.reshape(B * F)
    bias_vec = jnp.broadcast_to(bias.astype(jnp.float32), (_L,))
    out = _build(B, F, V)(idx_t, table, bias_vec)
    return out.reshape(B, U)


# speedup vs baseline: 3.0679x; 1.0109x over previous
import functools

import jax
import jax.numpy as jnp
from jax import lax
from jax.experimental import pallas as pl
from jax.experimental.pallas import tpu as pltpu
from jax.experimental.pallas import tpu_sc as plsc

_NC = 2
_NS = 16
_L = 16
_NW = _NC * _NS


@functools.lru_cache(maxsize=None)
def _build(B, F, V, VS):
    rows_w = B // _NW
    bf_w = rows_w * F
    n_chunks = rows_w // _L

    mesh = plsc.VectorSubcoreMesh(
        core_axis_name="c", subcore_axis_name="s",
        num_cores=_NC, num_subcores=_NS,
    )

    @functools.partial(
        pl.kernel,
        out_type=jax.ShapeDtypeStruct((B,), jnp.float32),
        mesh=mesh,
        scratch_types=[
            pltpu.VMEM((bf_w,), jnp.int32),
            pltpu.VMEM((bf_w,), jnp.float32),
            pltpu.VMEM((rows_w,), jnp.float32),
            pltpu.VMEM((_L,), jnp.float32),
            pltpu.SemaphoreType.DMA,
            pltpu.SemaphoreType.DMA,
        ],
    )
    def launch(idx_hbm, table_hbm, bias_hbm, out_hbm,
               idx_v, vals_v, out_v, bias_v, sem_i, sem_g):
        wid = lax.axis_index("s") * _NC + lax.axis_index("c")
        idx_copies = [
            pltpu.async_copy(
                idx_hbm.at[pl.ds(f * B + wid * rows_w, rows_w)],
                idx_v.at[pl.ds(f * rows_w, rows_w)],
                sem_i,
            )
            for f in range(F)
        ]
        pltpu.sync_copy(bias_hbm, bias_v)
        for c in idx_copies:
            c.wait()

        gathers = []
        for f in range(F):
            for j in range(rows_w // _L):
                sl = pl.ds(f * rows_w + j * _L, _L)
                idx_v[sl] = idx_v[sl] + f * VS
            gathers.append(pltpu.async_copy(
                table_hbm.at[idx_v.at[pl.ds(f * rows_w, rows_w)]],
                vals_v.at[pl.ds(f * rows_w, rows_w)],
                sem_g,
            ))
        for g in gathers:
            g.wait()

        bias_vec = bias_v[...]
        for c in range(n_chunks):
            acc = bias_vec
            for f in range(F):
                acc = acc + vals_v[pl.ds(f * rows_w + c * _L, _L)]
            out_v[pl.ds(c * _L, _L)] = acc

        pltpu.sync_copy(out_v, out_hbm.at[pl.ds(wid * rows_w, rows_w)])

    return launch


def kernel(indices, weights, bias):
    B, F = indices.shape
    _, V, U = weights.shape
    VS = V
    table = lax.optimization_barrier(weights.squeeze(-1)).reshape(F * V)
    idx_t = indices.T.reshape(B * F)
    bias_vec = jnp.broadcast_to(bias.astype(jnp.float32), (_L,))
    out = _build(B, F, V, VS)(idx_t, table, bias_vec)
    return out.reshape(B, U)
